# Initial kernel scaffold; baseline (speedup 1.0000x reference)
import jax
import jax.numpy as jnp
from jax.experimental import pallas as pl


def kernel(x_seq, edge_index, params):
    raise NotImplementedError("write your pallas kernel here")


# speedup vs baseline: 5.6390x; 5.6390x over previous
import functools

import jax
import jax.numpy as jnp
from jax import lax
from jax.experimental import pallas as pl
from jax.experimental.pallas import tpu as pltpu
from jax.experimental.pallas import tpu_sc as plsc

N = 10000
E = 640000
SEQ_LEN = 6
HORIZON = 3
NUM_FEATS = 2
OUTPUT_DIM = 2
HIDDEN = 64
NUM_LAYERS = 2

NC = 2
NS = 16
CHUNK = 128
NROW = 160
E_PAD = NC * NS * NROW * CHUNK
NACC = 10112
ZROWS = NACC // NS


def _make_seg_sum(w):
    mesh = plsc.VectorSubcoreMesh(core_axis_name="c", subcore_axis_name="s")

    @functools.partial(
        pl.kernel,
        out_type=jax.ShapeDtypeStruct((NC, NACC, w), jnp.float32),
        mesh=mesh,
        scratch_types=[
            pltpu.VMEM((NROW, CHUNK), jnp.int32),
            pltpu.VMEM((NROW, CHUNK), jnp.int32),
            pltpu.VMEM((CHUNK, w), jnp.float32),
            pltpu.VMEM_SHARED((NACC, w), jnp.float32),
            pltpu.SemaphoreType.DMA,
        ],
        compiler_params=pltpu.CompilerParams(use_tc_tiling_on_sc=False),
    )
    def k(table, srcp, dstp, zeros, out, src_v, dst_v, rows_v, acc, sem):
        cid = lax.axis_index("c")
        sid = lax.axis_index("s")
        wid = sid * NC + cid
        pltpu.sync_copy(zeros, acc.at[pl.ds(sid * ZROWS, ZROWS)])
        pltpu.sync_copy(srcp.at[pl.ds(wid * NROW, NROW)], src_v)
        pltpu.sync_copy(dstp.at[pl.ds(wid * NROW, NROW)], dst_v)
        plsc.subcore_barrier()

        def body(i, carry):
            pltpu.async_copy(table.at[src_v.at[i]], rows_v, sem).wait()
            pltpu.sync_copy(rows_v, acc.at[dst_v.at[i]], add=True)
            return carry

        lax.fori_loop(0, NROW, body, 0)
        plsc.subcore_barrier()
        pltpu.sync_copy(acc.at[pl.ds(sid * ZROWS, ZROWS)],
                        out.at[cid, pl.ds(sid * ZROWS, ZROWS)])

    return k


_SEG_SUM = {}


def _agg(table, srcp, dstp, inv_deg):
    w = table.shape[1]
    if w not in _SEG_SUM:
        _SEG_SUM[w] = _make_seg_sum(w)
    zeros = jnp.zeros((ZROWS, w), jnp.float32)
    p = _SEG_SUM[w](table, srcp, dstp, zeros)
    s = p[0, :N] + p[1, :N]
    return s if inv_deg is None else s * inv_deg


def _pad16(x):
    return jnp.concatenate(
        [x, jnp.zeros((N, 16 - x.shape[1]), jnp.float32)], axis=1)


def _cell(x, h, ax, srcp, dstp, inv_deg, p, h_zero):
    d = x.shape[1]
    if h_zero:
        ah = jnp.zeros((N, HIDDEN), jnp.float32)
    else:
        ah = _agg(h, srcp, dstp, inv_deg)
    xh = jnp.concatenate([x, h], axis=1)
    a1 = jnp.concatenate([ax, ah], axis=1)
    ru = jax.nn.sigmoid(xh @ p['Ws_ru'] + a1 @ p['Wn_ru'] + p['b_ru'])
    r, u = jnp.split(ru, 2, axis=1)
    rh = r * h
    if h_zero:
        arh = ah
    else:
        arh = _agg(rh, srcp, dstp, inv_deg)
    xc = jnp.concatenate([x, rh], axis=1)
    axc = jnp.concatenate([ax, arh], axis=1)
    c = jnp.tanh(xc @ p['Ws_c'] + axc @ p['Wn_c'] + p['b_c'])
    return u * h + (1.0 - u) * c


def kernel(x_seq, edge_index, params):
    src = edge_index[0]
    dst = edge_index[1]
    pad = E_PAD - E
    srcp = jnp.concatenate([src, jnp.zeros((pad,), jnp.int32)]).reshape(-1, CHUNK)
    dstp = jnp.concatenate([dst, jnp.full((pad,), N, jnp.int32)]).reshape(-1, CHUNK)

    ones = jnp.ones((N, 16), jnp.float32)
    deg = _agg(ones, srcp, dstp, None)[:, 0:1]
    inv_deg = 1.0 / jnp.clip(deg, 1.0, None)

    xflat = jnp.moveaxis(x_seq, 0, 1).reshape(N, SEQ_LEN * NUM_FEATS)
    ax_enc = _agg(_pad16(xflat), srcp, dstp, inv_deg)

    h = [jnp.zeros((N, HIDDEN), jnp.float32) for _ in range(NUM_LAYERS)]
    for t in range(SEQ_LEN):
        inp = x_seq[t]
        ax = ax_enc[:, t * NUM_FEATS:(t + 1) * NUM_FEATS]
        for l in range(NUM_LAYERS):
            h[l] = _cell(inp, h[l], ax, srcp, dstp, inv_deg,
                         params['enc'][l], h_zero=(t == 0))
            inp = h[l]
            if l + 1 < NUM_LAYERS:
                ax = _agg(inp, srcp, dstp, inv_deg)

    dec_in = jnp.zeros((N, OUTPUT_DIM), jnp.float32)
    outs = []
    for t in range(HORIZON):
        inp = dec_in
        if t == 0:
            ax = jnp.zeros((N, OUTPUT_DIM), jnp.float32)
        else:
            ax = _agg(_pad16(inp), srcp, dstp, inv_deg)[:, :OUTPUT_DIM]
        for l in range(NUM_LAYERS):
            h[l] = _cell(inp, h[l], ax, srcp, dstp, inv_deg,
                         params['dec'][l], h_zero=False)
            inp = h[l]
            if l + 1 < NUM_LAYERS:
                ax = _agg(inp, srcp, dstp, inv_deg)
        dec_in = inp @ params['W_out'] + params['b_out']
        outs.append(dec_in)
    return jnp.stack(outs)


# speedup vs baseline: 16.7341x; 2.9676x over previous
import functools

import jax
import jax.numpy as jnp
from jax import lax
from jax.experimental import pallas as pl
from jax.experimental.pallas import tpu as pltpu
from jax.experimental.pallas import tpu_sc as plsc

N = 10000
E = 640000
SEQ_LEN = 6
HORIZON = 3
NUM_FEATS = 2
OUTPUT_DIM = 2
HIDDEN = 64
NUM_LAYERS = 2

NC = 2
NS = 16
CHUNK = 128
NROWT = 320
E_PAD = NS * NROWT * CHUNK
NACC = 10112
ZROWS = NACC // NS


def _make_seg_sum(w):
    hw = w // 2
    mesh = plsc.VectorSubcoreMesh(core_axis_name="c", subcore_axis_name="s")

    @functools.partial(
        pl.kernel,
        out_type=jax.ShapeDtypeStruct((NC, NACC, hw), jnp.float32),
        mesh=mesh,
        scratch_types=[
            pltpu.VMEM((NROWT, CHUNK), jnp.int32),
            pltpu.VMEM((NROWT, CHUNK), jnp.int32),
            pltpu.VMEM((CHUNK, hw), jnp.float32),
            pltpu.VMEM((CHUNK, hw), jnp.float32),
            pltpu.VMEM_SHARED((NACC, hw), jnp.float32),
            pltpu.VMEM_SHARED((NACC, hw), jnp.float32),
            pltpu.SemaphoreType.DMA,
            pltpu.SemaphoreType.DMA,
        ],
        compiler_params=pltpu.CompilerParams(use_tc_tiling_on_sc=False),
    )
    def k(table, srcp, dstp, zeros, out, src_v, dst_v, rows0, rows1, acc,
          tbl, sem0, sem1):
        cid = lax.axis_index("c")
        sid = lax.axis_index("s")
        pltpu.sync_copy(zeros, acc.at[pl.ds(sid * ZROWS, ZROWS)])
        pltpu.sync_copy(table.at[cid, pl.ds(sid * ZROWS, ZROWS)],
                        tbl.at[pl.ds(sid * ZROWS, ZROWS)])
        pltpu.sync_copy(srcp.at[pl.ds(sid * NROWT, NROWT)], src_v)
        pltpu.sync_copy(dstp.at[pl.ds(sid * NROWT, NROWT)], dst_v)
        plsc.subcore_barrier()

        pltpu.async_copy(tbl.at[src_v.at[0]], rows0, sem0)

        def body(j, carry):
            i0 = 2 * j
            i1 = 2 * j + 1
            pltpu.async_copy(tbl.at[src_v.at[i1]], rows1, sem1)
            pltpu.make_async_copy(tbl.at[src_v.at[i0]], rows0, sem0).wait()
            pltpu.sync_copy(rows0, acc.at[dst_v.at[i0]], add=True)
            i2 = jnp.minimum(i0 + 2, NROWT - 1)
            pltpu.async_copy(tbl.at[src_v.at[i2]], rows0, sem0)
            pltpu.make_async_copy(tbl.at[src_v.at[i1]], rows1, sem1).wait()
            pltpu.sync_copy(rows1, acc.at[dst_v.at[i1]], add=True)
            return carry

        lax.fori_loop(0, NROWT // 2, body, 0)
        pltpu.make_async_copy(tbl.at[src_v.at[NROWT - 1]], rows0, sem0).wait()
        plsc.subcore_barrier()
        pltpu.sync_copy(acc.at[pl.ds(sid * ZROWS, ZROWS)],
                        out.at[cid, pl.ds(sid * ZROWS, ZROWS)])

    return k


_SEG_SUM = {}


def _agg(table, srcp, dstp, inv_deg):
    w = 64
    if w not in _SEG_SUM:
        _SEG_SUM[w] = _make_seg_sum(w)
    zeros = jnp.zeros((ZROWS, w // 2), jnp.float32)
    table_p = jnp.zeros((NACC, w), jnp.float32)
    table_p = lax.dynamic_update_slice(table_p, table, (0, 0))
    table_2 = jnp.moveaxis(table_p.reshape(NACC, NC, w // 2), 1, 0)
    p = _SEG_SUM[w](table_2, srcp, dstp, zeros)
    s = jnp.concatenate([p[0, :N], p[1, :N]], axis=1)
    return s if inv_deg is None else s * inv_deg


def _cell(x, h, ax, srcp, dstp, inv_deg, p, h_zero):
    d = x.shape[1]
    if h_zero:
        ah = jnp.zeros((N, HIDDEN), jnp.float32)
    else:
        ah = _agg(h, srcp, dstp, inv_deg)
    xh = jnp.concatenate([x, h], axis=1)
    a1 = jnp.concatenate([ax, ah], axis=1)
    ru = jax.nn.sigmoid(xh @ p['Ws_ru'] + a1 @ p['Wn_ru'] + p['b_ru'])
    r, u = jnp.split(ru, 2, axis=1)
    rh = r * h
    if h_zero:
        arh = ah
    else:
        arh = _agg(rh, srcp, dstp, inv_deg)
    xc = jnp.concatenate([x, rh], axis=1)
    axc = jnp.concatenate([ax, arh], axis=1)
    c = jnp.tanh(xc @ p['Ws_c'] + axc @ p['Wn_c'] + p['b_c'])
    return u * h + (1.0 - u) * c


def kernel(x_seq, edge_index, params):
    src = edge_index[0]
    dst = edge_index[1]
    pad = E_PAD - E
    srcp = jnp.concatenate([src, jnp.zeros((pad,), jnp.int32)]).reshape(-1, CHUNK)
    dstp = jnp.concatenate([dst, jnp.full((pad,), N, jnp.int32)]).reshape(-1, CHUNK)

    ones = jnp.ones((N, 16), jnp.float32)
    deg = _agg(ones, srcp, dstp, None)[:, 0:1]
    inv_deg = 1.0 / jnp.clip(deg, 1.0, None)

    xflat = jnp.moveaxis(x_seq, 0, 1).reshape(N, SEQ_LEN * NUM_FEATS)
    ax_enc = _agg(xflat, srcp, dstp, inv_deg)

    h = [jnp.zeros((N, HIDDEN), jnp.float32) for _ in range(NUM_LAYERS)]
    for t in range(SEQ_LEN):
        inp = x_seq[t]
        ax = ax_enc[:, t * NUM_FEATS:(t + 1) * NUM_FEATS]
        for l in range(NUM_LAYERS):
            h[l] = _cell(inp, h[l], ax, srcp, dstp, inv_deg,
                         params['enc'][l], h_zero=(t == 0))
            inp = h[l]
            if l + 1 < NUM_LAYERS:
                ax = _agg(inp, srcp, dstp, inv_deg)

    dec_in = jnp.zeros((N, OUTPUT_DIM), jnp.float32)
    outs = []
    for t in range(HORIZON):
        inp = dec_in
        if t == 0:
            ax = jnp.zeros((N, OUTPUT_DIM), jnp.float32)
        else:
            ax = _agg(inp, srcp, dstp, inv_deg)[:, :OUTPUT_DIM]
        for l in range(NUM_LAYERS):
            h[l] = _cell(inp, h[l], ax, srcp, dstp, inv_deg,
                         params['dec'][l], h_zero=False)
            inp = h[l]
            if l + 1 < NUM_LAYERS:
                ax = _agg(inp, srcp, dstp, inv_deg)
        dec_in = inp @ params['W_out'] + params['b_out']
        outs.append(dec_in)
    return jnp.stack(outs)


# speedup vs baseline: 17.3344x; 1.0359x over previous
import functools

import jax
import jax.numpy as jnp
from jax import lax
from jax.experimental import pallas as pl
from jax.experimental.pallas import tpu as pltpu
from jax.experimental.pallas import tpu_sc as plsc

N = 10000
E = 640000
SEQ_LEN = 6
HORIZON = 3
NUM_FEATS = 2
OUTPUT_DIM = 2
HIDDEN = 64
NUM_LAYERS = 2

NC = 2
NS = 16
CHUNK = 128
NROWT = 320
E_PAD = NS * NROWT * CHUNK
NACC = 10112
ZROWS = NACC // NS


def _make_seg_sum(w):
    hw = w // 2
    mesh = plsc.VectorSubcoreMesh(core_axis_name="c", subcore_axis_name="s")

    @functools.partial(
        pl.kernel,
        out_type=jax.ShapeDtypeStruct((NC, NACC, hw), jnp.float32),
        mesh=mesh,
        scratch_types=[
            pltpu.VMEM((NROWT, CHUNK), jnp.int32),
            pltpu.VMEM((NROWT, CHUNK), jnp.int32),
            pltpu.VMEM((CHUNK, hw), jnp.float32),
            pltpu.VMEM((CHUNK, hw), jnp.float32),
            pltpu.VMEM_SHARED((NACC, hw), jnp.float32),
            pltpu.VMEM_SHARED((NACC, hw), jnp.float32),
            pltpu.SemaphoreType.DMA,
            pltpu.SemaphoreType.DMA,
            pltpu.SemaphoreType.DMA,
        ],
        compiler_params=pltpu.CompilerParams(use_tc_tiling_on_sc=False),
    )
    def k(table, srcp, dstp, zeros, out, src_v, dst_v, rows0, rows1, acc,
          tbl, sem0, sem1, semp):
        cid = lax.axis_index("c")
        sid = lax.axis_index("s")
        pltpu.async_copy(zeros, acc.at[pl.ds(sid * ZROWS, ZROWS)], semp)
        pltpu.async_copy(table.at[cid, pl.ds(sid * ZROWS, ZROWS)],
                         tbl.at[pl.ds(sid * ZROWS, ZROWS)], semp)
        pltpu.async_copy(srcp.at[pl.ds(sid * NROWT, NROWT)], src_v, semp)
        pltpu.async_copy(dstp.at[pl.ds(sid * NROWT, NROWT)], dst_v, semp)
        pltpu.make_async_copy(zeros, acc.at[pl.ds(sid * ZROWS, ZROWS)], semp).wait()
        pltpu.make_async_copy(table.at[cid, pl.ds(sid * ZROWS, ZROWS)],
                              tbl.at[pl.ds(sid * ZROWS, ZROWS)], semp).wait()
        pltpu.make_async_copy(srcp.at[pl.ds(sid * NROWT, NROWT)], src_v, semp).wait()
        pltpu.make_async_copy(dstp.at[pl.ds(sid * NROWT, NROWT)], dst_v, semp).wait()
        plsc.subcore_barrier()

        pltpu.async_copy(tbl.at[src_v.at[0]], rows0, sem0)

        def body(j, carry):
            i0 = 2 * j
            i1 = 2 * j + 1
            pltpu.async_copy(tbl.at[src_v.at[i1]], rows1, sem1)
            pltpu.make_async_copy(tbl.at[src_v.at[i0]], rows0, sem0).wait()
            pltpu.sync_copy(rows0, acc.at[dst_v.at[i0]], add=True)
            i2 = jnp.minimum(i0 + 2, NROWT - 1)
            pltpu.async_copy(tbl.at[src_v.at[i2]], rows0, sem0)
            pltpu.make_async_copy(tbl.at[src_v.at[i1]], rows1, sem1).wait()
            pltpu.sync_copy(rows1, acc.at[dst_v.at[i1]], add=True)
            return carry

        lax.fori_loop(0, NROWT // 2, body, 0)
        pltpu.make_async_copy(tbl.at[src_v.at[NROWT - 1]], rows0, sem0).wait()
        plsc.subcore_barrier()
        pltpu.sync_copy(acc.at[pl.ds(sid * ZROWS, ZROWS)],
                        out.at[cid, pl.ds(sid * ZROWS, ZROWS)])

    return k


_SEG_SUM = {}


def _agg(table, srcp, dstp, inv_deg):
    w = 64
    if w not in _SEG_SUM:
        _SEG_SUM[w] = _make_seg_sum(w)
    zeros = jnp.zeros((ZROWS, w // 2), jnp.float32)
    table_p = jnp.zeros((NACC, w), jnp.float32)
    table_p = lax.dynamic_update_slice(table_p, table, (0, 0))
    table_2 = jnp.moveaxis(table_p.reshape(NACC, NC, w // 2), 1, 0)
    p = _SEG_SUM[w](table_2, srcp, dstp, zeros)
    s = jnp.concatenate([p[0, :N], p[1, :N]], axis=1)
    return s if inv_deg is None else s * inv_deg


def _cell(x, h, ax, srcp, dstp, inv_deg, p, h_zero):
    d = x.shape[1]
    if h_zero:
        ah = jnp.zeros((N, HIDDEN), jnp.float32)
    else:
        ah = _agg(h, srcp, dstp, inv_deg)
    xh = jnp.concatenate([x, h], axis=1)
    a1 = jnp.concatenate([ax, ah], axis=1)
    ru = jax.nn.sigmoid(xh @ p['Ws_ru'] + a1 @ p['Wn_ru'] + p['b_ru'])
    r, u = jnp.split(ru, 2, axis=1)
    rh = r * h
    if h_zero:
        arh = ah
    else:
        arh = _agg(rh, srcp, dstp, inv_deg)
    xc = jnp.concatenate([x, rh], axis=1)
    axc = jnp.concatenate([ax, arh], axis=1)
    c = jnp.tanh(xc @ p['Ws_c'] + axc @ p['Wn_c'] + p['b_c'])
    return u * h + (1.0 - u) * c


def kernel(x_seq, edge_index, params):
    src = edge_index[0]
    dst = edge_index[1]
    pad = E_PAD - E
    srcp = jnp.concatenate([src, jnp.zeros((pad,), jnp.int32)]).reshape(-1, CHUNK)
    dstp = jnp.concatenate([dst, jnp.full((pad,), N, jnp.int32)]).reshape(-1, CHUNK)

    xflat = jnp.moveaxis(x_seq, 0, 1).reshape(N, SEQ_LEN * NUM_FEATS)
    xdeg = jnp.concatenate([xflat, jnp.ones((N, 1), jnp.float32)], axis=1)
    s0 = _agg(xdeg, srcp, dstp, None)
    deg = s0[:, SEQ_LEN * NUM_FEATS:SEQ_LEN * NUM_FEATS + 1]
    inv_deg = 1.0 / jnp.clip(deg, 1.0, None)
    ax_enc = s0[:, :SEQ_LEN * NUM_FEATS] * inv_deg

    h = [jnp.zeros((N, HIDDEN), jnp.float32) for _ in range(NUM_LAYERS)]
    for t in range(SEQ_LEN):
        inp = x_seq[t]
        ax = ax_enc[:, t * NUM_FEATS:(t + 1) * NUM_FEATS]
        for l in range(NUM_LAYERS):
            h[l] = _cell(inp, h[l], ax, srcp, dstp, inv_deg,
                         params['enc'][l], h_zero=(t == 0))
            inp = h[l]
            if l + 1 < NUM_LAYERS:
                ax = _agg(inp, srcp, dstp, inv_deg)

    dec_in = jnp.zeros((N, OUTPUT_DIM), jnp.float32)
    outs = []
    for t in range(HORIZON):
        inp = dec_in
        if t == 0:
            ax = jnp.zeros((N, OUTPUT_DIM), jnp.float32)
        else:
            ax = _agg(inp, srcp, dstp, inv_deg)[:, :OUTPUT_DIM]
        for l in range(NUM_LAYERS):
            h[l] = _cell(inp, h[l], ax, srcp, dstp, inv_deg,
                         params['dec'][l], h_zero=False)
            inp = h[l]
            if l + 1 < NUM_LAYERS:
                ax = _agg(inp, srcp, dstp, inv_deg)
        dec_in = inp @ params['W_out'] + params['b_out']
        outs.append(dec_in)
    return jnp.stack(outs)


# speedup vs baseline: 19.0577x; 1.0994x over previous
import functools

import jax
import jax.numpy as jnp
from jax import lax
from jax.experimental import pallas as pl
from jax.experimental.pallas import tpu as pltpu
from jax.experimental.pallas import tpu_sc as plsc

N = 10000
E = 640000
SEQ_LEN = 6
HORIZON = 3
NUM_FEATS = 2
OUTPUT_DIM = 2
HIDDEN = 64
NUM_LAYERS = 2

NC = 2
NS = 16
CHUNK = 128
NROWT = 320
E_PAD = NS * NROWT * CHUNK
NACC = 10112
ZROWS = NACC // NS


def _make_seg_sum(w):
    hw = w // 2
    mesh = plsc.VectorSubcoreMesh(core_axis_name="c", subcore_axis_name="s")

    @functools.partial(
        pl.kernel,
        out_type=jax.ShapeDtypeStruct((NC, NACC, hw), jnp.float32),
        mesh=mesh,
        scratch_types=[
            pltpu.VMEM((NROWT, CHUNK), jnp.int32),
            pltpu.VMEM((NROWT, CHUNK), jnp.int32),
            pltpu.VMEM((CHUNK, hw), jnp.float32),
            pltpu.VMEM((CHUNK, hw), jnp.float32),
            pltpu.VMEM_SHARED((NACC, hw), jnp.float32),
            pltpu.VMEM_SHARED((NACC, hw), jnp.float32),
            pltpu.SemaphoreType.DMA,
            pltpu.SemaphoreType.DMA,
            pltpu.SemaphoreType.DMA,
        ],
        compiler_params=pltpu.CompilerParams(use_tc_tiling_on_sc=False),
    )
    def k(table, srcp, dstp, zeros, out, src_v, dst_v, rows0, rows1, acc,
          tbl, sem0, sem1, semp):
        cid = lax.axis_index("c")
        sid = lax.axis_index("s")
        pltpu.async_copy(zeros, acc.at[pl.ds(sid * ZROWS, ZROWS)], semp)
        pltpu.async_copy(table.at[cid, pl.ds(sid * ZROWS, ZROWS)],
                         tbl.at[pl.ds(sid * ZROWS, ZROWS)], semp)
        pltpu.async_copy(srcp.at[pl.ds(sid * NROWT, NROWT)], src_v, semp)
        pltpu.async_copy(dstp.at[pl.ds(sid * NROWT, NROWT)], dst_v, semp)
        pltpu.make_async_copy(zeros, acc.at[pl.ds(sid * ZROWS, ZROWS)], semp).wait()
        pltpu.make_async_copy(table.at[cid, pl.ds(sid * ZROWS, ZROWS)],
                              tbl.at[pl.ds(sid * ZROWS, ZROWS)], semp).wait()
        pltpu.make_async_copy(srcp.at[pl.ds(sid * NROWT, NROWT)], src_v, semp).wait()
        pltpu.make_async_copy(dstp.at[pl.ds(sid * NROWT, NROWT)], dst_v, semp).wait()
        plsc.subcore_barrier()

        pltpu.async_copy(tbl.at[src_v.at[0]], rows0, sem0)

        def body(j, carry):
            i0 = 2 * j
            i1 = 2 * j + 1
            pltpu.async_copy(tbl.at[src_v.at[i1]], rows1, sem1)
            pltpu.make_async_copy(tbl.at[src_v.at[i0]], rows0, sem0).wait()
            pltpu.sync_copy(rows0, acc.at[dst_v.at[i0]], add=True)
            i2 = jnp.minimum(i0 + 2, NROWT - 1)
            pltpu.async_copy(tbl.at[src_v.at[i2]], rows0, sem0)
            pltpu.make_async_copy(tbl.at[src_v.at[i1]], rows1, sem1).wait()
            pltpu.sync_copy(rows1, acc.at[dst_v.at[i1]], add=True)
            return carry

        lax.fori_loop(0, NROWT // 2, body, 0)
        pltpu.make_async_copy(tbl.at[src_v.at[NROWT - 1]], rows0, sem0).wait()
        plsc.subcore_barrier()
        pltpu.sync_copy(acc.at[pl.ds(sid * ZROWS, ZROWS)],
                        out.at[cid, pl.ds(sid * ZROWS, ZROWS)])

    return k


_SEG_SUM = {}


def _agg(table, srcp, dstp, inv_deg):
    w = 64
    if w not in _SEG_SUM:
        _SEG_SUM[w] = _make_seg_sum(w)
    zeros = jnp.zeros((ZROWS, w // 2), jnp.float32)
    table_p = jnp.zeros((NACC, w), jnp.float32)
    table_p = lax.dynamic_update_slice(table_p, table, (0, 0))
    table_2 = jnp.moveaxis(table_p.reshape(NACC, NC, w // 2), 1, 0)
    p = _SEG_SUM[w](table_2, srcp, dstp, zeros)
    s = jnp.concatenate([p[0, :N], p[1, :N]], axis=1)
    return s if inv_deg is None else s * inv_deg


_DENSE = {}


def _dense(kind, d):
    key = (kind, d)
    if key in _DENSE:
        return _DENSE[key]
    f32 = jnp.float32
    o2 = [jax.ShapeDtypeStruct((N, HIDDEN), f32)] * 2

    if kind == 'a':
        def body(x, h, ax, ah, wsx, wsh, wnx, wnh, b, rh_o, u_o):
            z = (x[...] @ wsx[...] + h[...] @ wsh[...]
                 + ax[...] @ wnx[...] + ah[...] @ wnh[...] + b[...])
            ru = jax.nn.sigmoid(z)
            rh_o[...] = ru[:, :HIDDEN] * h[...]
            u_o[...] = ru[:, HIDDEN:]
        f = pl.pallas_call(body, out_shape=o2)
    elif kind == 'a0':
        def body(x, ax, wrx, wnrx, br, wcx, wncx, bc, h_o):
            u = jax.nn.sigmoid(
                x[...] @ wrx[...] + ax[...] @ wnrx[...] + br[...])[:, HIDDEN:]
            c = jnp.tanh(x[...] @ wcx[...] + ax[...] @ wncx[...] + bc[...])
            h_o[...] = (1.0 - u) * c
        f = pl.pallas_call(
            body, out_shape=jax.ShapeDtypeStruct((N, HIDDEN), f32))
    elif kind == 'b':
        def body(x, rh, ax, arh, u, h, wsx, wsh, wnx, wnh, b, h_o):
            z = (x[...] @ wsx[...] + rh[...] @ wsh[...]
                 + ax[...] @ wnx[...] + arh[...] @ wnh[...] + b[...])
            c = jnp.tanh(z)
            h_o[...] = u[...] * h[...] + (1.0 - u[...]) * c
        f = pl.pallas_call(
            body, out_shape=jax.ShapeDtypeStruct((N, HIDDEN), f32))
    else:
        def body(x, rh, ax, arh, u, h, wsx, wsh, wnx, wnh, b, wo, bo,
                 h_o, y_o):
            z = (x[...] @ wsx[...] + rh[...] @ wsh[...]
                 + ax[...] @ wnx[...] + arh[...] @ wnh[...] + b[...])
            c = jnp.tanh(z)
            hn = u[...] * h[...] + (1.0 - u[...]) * c
            h_o[...] = hn
            y_o[...] = hn @ wo[...] + bo[...]
        f = pl.pallas_call(
            body, out_shape=[jax.ShapeDtypeStruct((N, HIDDEN), f32),
                             jax.ShapeDtypeStruct((N, OUTPUT_DIM), f32)])
    _DENSE[key] = f
    return f


def _cell(x, h, ax, srcp, dstp, inv_deg, p, h_zero, wout=None, bout=None):
    d = x.shape[1]
    br = p['b_ru'].reshape(1, -1)
    bc = p['b_c'].reshape(1, -1)
    if h_zero:
        hn = _dense('a0', d)(x, ax, p['Ws_ru'][:d], p['Wn_ru'][:d], br,
                             p['Ws_c'][:d], p['Wn_c'][:d], bc)
        return hn, None
    ah = _agg(h, srcp, dstp, inv_deg)
    rh, u = _dense('a', d)(x, h, ax, ah,
                           p['Ws_ru'][:d], p['Ws_ru'][d:],
                           p['Wn_ru'][:d], p['Wn_ru'][d:], br)
    arh = _agg(rh, srcp, dstp, inv_deg)
    args = (x, rh, ax, arh, u, h,
            p['Ws_c'][:d], p['Ws_c'][d:], p['Wn_c'][:d], p['Wn_c'][d:], bc)
    if wout is None:
        return _dense('b', d)(*args), None
    hn, y = _dense('bp', d)(*args, wout, bout.reshape(1, -1))
    return hn, y


def kernel(x_seq, edge_index, params):
    src = edge_index[0]
    dst = edge_index[1]
    pad = E_PAD - E
    srcp = jnp.concatenate([src, jnp.zeros((pad,), jnp.int32)]).reshape(-1, CHUNK)
    dstp = jnp.concatenate([dst, jnp.full((pad,), N, jnp.int32)]).reshape(-1, CHUNK)

    xflat = jnp.moveaxis(x_seq, 0, 1).reshape(N, SEQ_LEN * NUM_FEATS)
    xdeg = jnp.concatenate([xflat, jnp.ones((N, 1), jnp.float32)], axis=1)
    s0 = _agg(xdeg, srcp, dstp, None)
    deg = s0[:, SEQ_LEN * NUM_FEATS:SEQ_LEN * NUM_FEATS + 1]
    inv_deg = 1.0 / jnp.clip(deg, 1.0, None)
    ax_enc = s0[:, :SEQ_LEN * NUM_FEATS] * inv_deg

    h = [jnp.zeros((N, HIDDEN), jnp.float32) for _ in range(NUM_LAYERS)]
    for t in range(SEQ_LEN):
        inp = x_seq[t]
        ax = ax_enc[:, t * NUM_FEATS:(t + 1) * NUM_FEATS]
        for l in range(NUM_LAYERS):
            h[l], _ = _cell(inp, h[l], ax, srcp, dstp, inv_deg,
                            params['enc'][l], h_zero=(t == 0))
            inp = h[l]
            if l + 1 < NUM_LAYERS:
                ax = _agg(inp, srcp, dstp, inv_deg)

    dec_in = jnp.zeros((N, OUTPUT_DIM), jnp.float32)
    outs = []
    for t in range(HORIZON):
        inp = dec_in
        if t == 0:
            ax = jnp.zeros((N, OUTPUT_DIM), jnp.float32)
        else:
            ax = _agg(inp, srcp, dstp, inv_deg)[:, :OUTPUT_DIM]
        for l in range(NUM_LAYERS):
            last = l + 1 == NUM_LAYERS
            h[l], y = _cell(inp, h[l], ax, srcp, dstp, inv_deg,
                            params['dec'][l], h_zero=False,
                            wout=params['W_out'] if last else None,
                            bout=params['b_out'] if last else None)
            inp = h[l]
            if not last:
                ax = _agg(inp, srcp, dstp, inv_deg)
        outs.append(y)
    return jnp.stack(outs)
